# baseline (speedup 1.0000x reference)
import functools

import jax
import jax.numpy as jnp
from jax import lax
from jax.experimental import pallas as pl
from jax.experimental.pallas import tpu as pltpu

_D = 128
_RB = 1024


def _tc_permute(x_rows, idx):
    n = x_rows.shape[0]
    grid = n // _RB

    def body(idx_ref, x_ref, o_ref):
        cols = jax.lax.broadcasted_iota(jnp.int32, (_D, _D), 0)
        onehot = (idx_ref[...] == cols).astype(jnp.float32)
        o_ref[...] = jax.lax.dot_general(
            x_ref[...], onehot,
            dimension_numbers=(((1,), (0,)), ((), ())),
            preferred_element_type=jnp.float32,
        )

    return pl.pallas_call(
        body,
        grid=(grid,),
        in_specs=[
            pl.BlockSpec((1, _D), lambda i: (0, 0)),
            pl.BlockSpec((_RB, _D), lambda i: (i, 0)),
        ],
        out_specs=pl.BlockSpec((_RB, _D), lambda i: (i, 0)),
        out_shape=jax.ShapeDtypeStruct((n, _D), jnp.float32),
    )(idx.reshape(1, _D), x_rows)


def kernel(x, indexes):
    b, s, d = x.shape
    x_rows = x.reshape(b * s, d)
    idx = indexes.astype(jnp.int32)
    out = _tc_permute(x_rows, idx)
    return out.reshape(b, s, d)


# speedup vs baseline: 1.2226x; 1.2226x over previous
import functools

import jax
import jax.numpy as jnp
from jax import lax
from jax.experimental import pallas as pl
from jax.experimental.pallas import tpu as pltpu

_D = 128
_RB = 8192


def _tc_permute(x_rows, idx):
    n = x_rows.shape[0]
    grid = n // _RB

    def body(idx_ref, x_ref, o_ref):
        cols = jax.lax.broadcasted_iota(jnp.int32, (_D, _D), 0)
        onehot = (idx_ref[...] == cols).astype(jnp.float32)
        o_ref[...] = jax.lax.dot_general(
            x_ref[...], onehot,
            dimension_numbers=(((1,), (0,)), ((), ())),
            preferred_element_type=jnp.float32,
        )

    return pl.pallas_call(
        body,
        grid=(grid,),
        in_specs=[
            pl.BlockSpec((1, _D), lambda i: (0, 0)),
            pl.BlockSpec((_RB, _D), lambda i: (i, 0)),
        ],
        out_specs=pl.BlockSpec((_RB, _D), lambda i: (i, 0)),
        out_shape=jax.ShapeDtypeStruct((n, _D), jnp.float32),
    )(idx.reshape(1, _D), x_rows)


def kernel(x, indexes):
    b, s, d = x.shape
    x_rows = x.reshape(b * s, d)
    idx = indexes.astype(jnp.int32)
    out = _tc_permute(x_rows, idx)
    return out.reshape(b, s, d)
